# baseline (speedup 1.0000x reference)
import functools

import jax
import jax.numpy as jnp
from jax import lax
from jax.experimental import pallas as pl
from jax.experimental.pallas import tpu as pltpu
from jax.experimental.pallas import tpu_sc as plsc

C = 96
CF = 128
K = 9
TILE = 784
NW = 32
CH = 7


def _stem_body(p_ref, w_ref, b_ref, f_ref, xn_ref):
    p = p_ref[0]
    f = jax.nn.gelu(jnp.dot(p, w_ref[...], preferred_element_type=jnp.float32)
                    + b_ref[...])
    f_ref[0] = jnp.pad(f, ((0, 0), (0, CF - C)))
    nrm = jnp.sqrt(jnp.sum(f * f, axis=1, keepdims=True))
    xn_ref[0] = f / (nrm + 1e-12)


def _knn_body(n_real, xt_ref, xf_ref, idx_ref):
    xt = xt_ref[...]
    xf = xf_ref[...]
    s = lax.dot_general(xt, xf, (((1,), (1,)), ((), ())),
                        preferred_element_type=jnp.float32)
    sq_t = jnp.sum(xt * xt, axis=1, keepdims=True)
    sq_f = jnp.sum(xf * xf, axis=1)
    d = sq_t - 2.0 * s + sq_f[None, :]
    rows = d.shape[0]
    big = jnp.int32(n_real + 1)
    inf = jnp.float32(jnp.inf)
    col = lax.broadcasted_iota(jnp.int32, d.shape, 1)
    lane = lax.broadcasted_iota(jnp.int32, (rows, K), 1)
    acc = jnp.zeros((rows, K), jnp.int32)
    for k in range(K):
        m = jnp.min(d, axis=1, keepdims=True)
        ik = jnp.min(jnp.where(d == m, col, big), axis=1)
        acc = jnp.where(lane == k, ik[:, None], acc)
        if k + 1 < K:
            d = jnp.where(col == ik[:, None], inf, d)
    idx_ref[...] = acc


def _mlp_body(f_ref, mx_ref, gw_ref, gb_ref, pw_ref, pb_ref,
              w1_ref, b1_ref, w2_ref, b2_ref, hw_ref, hb_ref, o_ref):
    f = f_ref[:, :C]
    mx = mx_ref[...] - f
    cat = jnp.concatenate([f, mx], axis=1)
    g = jax.nn.gelu(jnp.dot(cat, gw_ref[...],
                            preferred_element_type=jnp.float32) + gb_ref[...])
    f = f + jnp.dot(g, pw_ref[...], preferred_element_type=jnp.float32) \
          + pb_ref[...]
    h = jax.nn.gelu(jnp.dot(f, w1_ref[...],
                            preferred_element_type=jnp.float32) + b1_ref[...])
    f = f + jnp.dot(h, w2_ref[...], preferred_element_type=jnp.float32) \
          + b2_ref[...]
    o_ref[...] = jnp.dot(f, hw_ref[...],
                         preferred_element_type=jnp.float32) + hb_ref[...]


def _make_sc_gather_max(m_rows, nch):
    mpw = m_rows // NW
    rows = CH * K
    mesh = plsc.VectorSubcoreMesh(core_axis_name="c", subcore_axis_name="s")

    @functools.partial(
        pl.kernel, mesh=mesh,
        out_type=jax.ShapeDtypeStruct((m_rows * C,), jnp.float32),
        scratch_types=[
            pltpu.VMEM((nch, rows), jnp.int32),
            pltpu.VMEM((rows, CF), jnp.float32),
            pltpu.VMEM((rows, CF), jnp.float32),
            pltpu.VMEM((mpw * C,), jnp.float32),
            pltpu.SemaphoreType.DMA,
            pltpu.SemaphoreType.DMA,
        ],
    )
    def sc_gather_max(f_hbm, idx_hbm, out_hbm, idx_v, buf0, buf1, mxg_v,
                      sem0, sem1):
        wid = lax.axis_index("s") * 2 + lax.axis_index("c")
        pltpu.sync_copy(idx_hbm.at[wid], idx_v)
        bufs = (buf0, buf1)
        sems = (sem0, sem1)
        pltpu.async_copy(f_hbm.at[idx_v.at[0]], buf0, sem0)
        pltpu.async_copy(f_hbm.at[idx_v.at[1]], buf1, sem1)

        def compute(g, buf):
            for n in range(CH):
                node = g * CH + n
                for c in range(C // 16):
                    sl = pl.ds(c * 16, 16)
                    acc = buf[n * K, sl]
                    for k in range(1, K):
                        acc = jnp.maximum(acc, buf[n * K + k, sl])
                    mxg_v[pl.ds(node * C + c * 16, 16)] = acc

        def pair(i, carry):
            for b in range(2):
                g = i * 2 + b
                buf, sem = bufs[b], sems[b]
                pltpu.make_async_copy(f_hbm.at[idx_v.at[g]], buf, sem).wait()
                compute(g, buf)

                @pl.when(g + 2 < nch)
                def _():
                    pltpu.async_copy(f_hbm.at[idx_v.at[g + 2]], buf, sem)
            return carry

        lax.fori_loop(0, nch // 2, pair, 0)
        pltpu.sync_copy(mxg_v, out_hbm.at[pl.ds(wid * (mpw * C), mpw * C)])

    return sc_gather_max


def kernel(x, stem_w, stem_b, graph_w, graph_b, proj_w, proj_b,
           ffn_w1, ffn_b1, ffn_w2, ffn_b2, alpha,
           head_loop_w, head_loop_b, head_tad_w, head_tad_b):
    B = x.shape[0]
    hp = x.shape[2] // 4
    wp = x.shape[3] // 4
    n = hp * wp
    nt = n // TILE
    mpw = n // NW
    nch = mpw // CH

    patches = x[:, 0].reshape(B, hp, 4, wp, 4).transpose(0, 1, 3, 2, 4)
    patches = patches.reshape(B, n, 16)
    w_t = stem_w.reshape(C, 16).T

    f, xn = pl.pallas_call(
        _stem_body,
        grid=(B, nt),
        in_specs=[
            pl.BlockSpec((1, TILE, 16), lambda b, t: (b, t, 0)),
            pl.BlockSpec((16, C), lambda b, t: (0, 0)),
            pl.BlockSpec((1, C), lambda b, t: (0, 0)),
        ],
        out_specs=[
            pl.BlockSpec((1, TILE, CF), lambda b, t: (b, t, 0)),
            pl.BlockSpec((1, TILE, C), lambda b, t: (b, t, 0)),
        ],
        out_shape=[
            jax.ShapeDtypeStruct((B, n, CF), jnp.float32),
            jax.ShapeDtypeStruct((B, n, C), jnp.float32),
        ],
    )(patches, w_t, stem_b.reshape(1, C))

    knn_call = pl.pallas_call(
        functools.partial(_knn_body, n),
        grid=(nt,),
        in_specs=[
            pl.BlockSpec((TILE, C), lambda t: (t, 0)),
            pl.BlockSpec((n, C), lambda t: (0, 0)),
        ],
        out_specs=pl.BlockSpec((TILE, K), lambda t: (t, 0)),
        out_shape=jax.ShapeDtypeStruct((n, K), jnp.int32),
    )
    idx = [knn_call(xn[b], xn[b]) for b in range(B)]

    sc_call = _make_sc_gather_max(n, nch)
    mxg = [sc_call(f[b], idx[b].reshape(NW, nch, CH * K)).reshape(n, C)
           for b in range(B)]

    scale = jnp.sum(alpha, axis=1)
    hw = jnp.concatenate([head_loop_w, head_tad_w], axis=1) * scale[None, :]
    hb = jnp.concatenate([head_loop_b, head_tad_b]).reshape(1, 2)

    mlp_call = pl.pallas_call(
        _mlp_body,
        grid=(nt,),
        in_specs=[
            pl.BlockSpec((TILE, CF), lambda t: (t, 0)),
            pl.BlockSpec((TILE, C), lambda t: (t, 0)),
            pl.BlockSpec((2 * C, 2 * C), lambda t: (0, 0)),
            pl.BlockSpec((1, 2 * C), lambda t: (0, 0)),
            pl.BlockSpec((2 * C, C), lambda t: (0, 0)),
            pl.BlockSpec((1, C), lambda t: (0, 0)),
            pl.BlockSpec((C, 4 * C), lambda t: (0, 0)),
            pl.BlockSpec((1, 4 * C), lambda t: (0, 0)),
            pl.BlockSpec((4 * C, C), lambda t: (0, 0)),
            pl.BlockSpec((1, C), lambda t: (0, 0)),
            pl.BlockSpec((C, 2), lambda t: (0, 0)),
            pl.BlockSpec((1, 2), lambda t: (0, 0)),
        ],
        out_specs=pl.BlockSpec((TILE, 2), lambda t: (t, 0)),
        out_shape=jax.ShapeDtypeStruct((n, 2), jnp.float32),
    )
    out2 = jnp.stack([
        mlp_call(f[b], mxg[b], graph_w, graph_b.reshape(1, 2 * C), proj_w,
                 proj_b.reshape(1, C), ffn_w1, ffn_b1.reshape(1, 4 * C),
                 ffn_w2, ffn_b2.reshape(1, C), hw, hb)
        for b in range(B)
    ])

    return out2.transpose(0, 2, 1).reshape(B, 2, hp, wp)


# speedup vs baseline: 1.0730x; 1.0730x over previous
import functools

import jax
import jax.numpy as jnp
from jax import lax
from jax.experimental import pallas as pl
from jax.experimental.pallas import tpu as pltpu
from jax.experimental.pallas import tpu_sc as plsc

C = 96
CF = 128
K = 9
TILE = 448
NW = 32
CH = 7


def _stem_body(p_ref, w_ref, b_ref, f_ref, xn_ref):
    p = p_ref[0]
    f = jax.nn.gelu(jnp.dot(p, w_ref[...], preferred_element_type=jnp.float32)
                    + b_ref[...])
    f_ref[0] = jnp.pad(f, ((0, 0), (0, CF - C)))
    nrm = jnp.sqrt(jnp.sum(f * f, axis=1, keepdims=True))
    xn_ref[0] = f / (nrm + 1e-12)


def _knn_body(n_real, xt_ref, xf_ref, idx_ref):
    xt = xt_ref[...]
    xf = xf_ref[...]
    s = lax.dot_general(xt, xf, (((1,), (1,)), ((), ())),
                        preferred_element_type=jnp.float32)
    sq_t = jnp.sum(xt * xt, axis=1, keepdims=True)
    sq_f = jnp.sum(xf * xf, axis=1)
    d = sq_t - 2.0 * s + sq_f[None, :]
    rows = d.shape[0]
    big = jnp.int32(n_real + 1)
    inf = jnp.float32(jnp.inf)
    col = lax.broadcasted_iota(jnp.int32, d.shape, 1)
    lane = lax.broadcasted_iota(jnp.int32, (rows, K), 1)
    selfi = pl.program_id(0) * rows + lax.broadcasted_iota(
        jnp.int32, (rows, 1), 0)
    d = jnp.where(col == selfi, inf, d)
    acc = jnp.where(lane == 0, selfi, jnp.zeros((rows, K), jnp.int32))
    for k in range(1, K):
        m = jnp.min(d, axis=1, keepdims=True)
        ik = jnp.min(jnp.where(d == m, col, big), axis=1)
        acc = jnp.where(lane == k, ik[:, None], acc)
        if k + 1 < K:
            d = jnp.where(col == ik[:, None], inf, d)
    idx_ref[...] = acc


def _mlp_body(f_ref, mx_ref, gw_ref, gb_ref, pw_ref, pb_ref,
              w1_ref, b1_ref, w2_ref, b2_ref, hw_ref, hb_ref, o_ref):
    f = f_ref[:, :C]
    mx = mx_ref[...] - f
    cat = jnp.concatenate([f, mx], axis=1)
    g = jax.nn.gelu(jnp.dot(cat, gw_ref[...],
                            preferred_element_type=jnp.float32) + gb_ref[...])
    f = f + jnp.dot(g, pw_ref[...], preferred_element_type=jnp.float32) \
          + pb_ref[...]
    h = jax.nn.gelu(jnp.dot(f, w1_ref[...],
                            preferred_element_type=jnp.float32) + b1_ref[...])
    f = f + jnp.dot(h, w2_ref[...], preferred_element_type=jnp.float32) \
          + b2_ref[...]
    o_ref[...] = jnp.dot(f, hw_ref[...],
                         preferred_element_type=jnp.float32) + hb_ref[...]


def _make_sc_gather_max(m_rows, nch):
    mpw = m_rows // NW
    rows = CH * K
    mesh = plsc.VectorSubcoreMesh(core_axis_name="c", subcore_axis_name="s")

    @functools.partial(
        pl.kernel, mesh=mesh,
        out_type=jax.ShapeDtypeStruct((m_rows * C,), jnp.float32),
        scratch_types=[
            pltpu.VMEM((nch, rows), jnp.int32),
            pltpu.VMEM((rows, CF), jnp.float32),
            pltpu.VMEM((rows, CF), jnp.float32),
            pltpu.VMEM((mpw * C,), jnp.float32),
            pltpu.SemaphoreType.DMA,
            pltpu.SemaphoreType.DMA,
        ],
    )
    def sc_gather_max(f_hbm, idx_hbm, out_hbm, idx_v, buf0, buf1, mxg_v,
                      sem0, sem1):
        wid = lax.axis_index("s") * 2 + lax.axis_index("c")
        pltpu.sync_copy(idx_hbm.at[wid], idx_v)
        bufs = (buf0, buf1)
        sems = (sem0, sem1)
        pltpu.async_copy(f_hbm.at[idx_v.at[0]], buf0, sem0)
        pltpu.async_copy(f_hbm.at[idx_v.at[1]], buf1, sem1)

        def compute(g, buf):
            for n in range(CH):
                node = g * CH + n
                for c in range(C // 16):
                    sl = pl.ds(c * 16, 16)
                    acc = buf[n * K, sl]
                    for k in range(1, K):
                        acc = jnp.maximum(acc, buf[n * K + k, sl])
                    mxg_v[pl.ds(node * C + c * 16, 16)] = acc

        def pair(i, carry):
            for b in range(2):
                g = i * 2 + b
                buf, sem = bufs[b], sems[b]
                pltpu.make_async_copy(f_hbm.at[idx_v.at[g]], buf, sem).wait()
                compute(g, buf)

                @pl.when(g + 2 < nch)
                def _():
                    pltpu.async_copy(f_hbm.at[idx_v.at[g + 2]], buf, sem)
            return carry

        lax.fori_loop(0, nch // 2, pair, 0)
        pltpu.sync_copy(mxg_v, out_hbm.at[pl.ds(wid * (mpw * C), mpw * C)])

    return sc_gather_max


def kernel(x, stem_w, stem_b, graph_w, graph_b, proj_w, proj_b,
           ffn_w1, ffn_b1, ffn_w2, ffn_b2, alpha,
           head_loop_w, head_loop_b, head_tad_w, head_tad_b):
    B = x.shape[0]
    hp = x.shape[2] // 4
    wp = x.shape[3] // 4
    n = hp * wp
    nt = n // TILE
    mpw = n // NW
    nch = mpw // CH

    patches = x[:, 0].reshape(B, hp, 4, wp, 4).transpose(0, 1, 3, 2, 4)
    patches = patches.reshape(B, n, 16)
    w_t = stem_w.reshape(C, 16).T

    f, xn = pl.pallas_call(
        _stem_body,
        grid=(B, nt),
        in_specs=[
            pl.BlockSpec((1, TILE, 16), lambda b, t: (b, t, 0)),
            pl.BlockSpec((16, C), lambda b, t: (0, 0)),
            pl.BlockSpec((1, C), lambda b, t: (0, 0)),
        ],
        out_specs=[
            pl.BlockSpec((1, TILE, CF), lambda b, t: (b, t, 0)),
            pl.BlockSpec((1, TILE, C), lambda b, t: (b, t, 0)),
        ],
        out_shape=[
            jax.ShapeDtypeStruct((B, n, CF), jnp.float32),
            jax.ShapeDtypeStruct((B, n, C), jnp.float32),
        ],
    )(patches, w_t, stem_b.reshape(1, C))

    knn_call = pl.pallas_call(
        functools.partial(_knn_body, n),
        grid=(nt,),
        in_specs=[
            pl.BlockSpec((TILE, C), lambda t: (t, 0)),
            pl.BlockSpec((n, C), lambda t: (0, 0)),
        ],
        out_specs=pl.BlockSpec((TILE, K), lambda t: (t, 0)),
        out_shape=jax.ShapeDtypeStruct((n, K), jnp.int32),
    )
    idx = [knn_call(xn[b], xn[b]) for b in range(B)]

    sc_call = _make_sc_gather_max(n, nch)
    mxg = [sc_call(f[b], idx[b].reshape(NW, nch, CH * K)).reshape(n, C)
           for b in range(B)]

    scale = jnp.sum(alpha, axis=1)
    hw = jnp.concatenate([head_loop_w, head_tad_w], axis=1) * scale[None, :]
    hb = jnp.concatenate([head_loop_b, head_tad_b]).reshape(1, 2)

    mlp_call = pl.pallas_call(
        _mlp_body,
        grid=(nt,),
        in_specs=[
            pl.BlockSpec((TILE, CF), lambda t: (t, 0)),
            pl.BlockSpec((TILE, C), lambda t: (t, 0)),
            pl.BlockSpec((2 * C, 2 * C), lambda t: (0, 0)),
            pl.BlockSpec((1, 2 * C), lambda t: (0, 0)),
            pl.BlockSpec((2 * C, C), lambda t: (0, 0)),
            pl.BlockSpec((1, C), lambda t: (0, 0)),
            pl.BlockSpec((C, 4 * C), lambda t: (0, 0)),
            pl.BlockSpec((1, 4 * C), lambda t: (0, 0)),
            pl.BlockSpec((4 * C, C), lambda t: (0, 0)),
            pl.BlockSpec((1, C), lambda t: (0, 0)),
            pl.BlockSpec((C, 2), lambda t: (0, 0)),
            pl.BlockSpec((1, 2), lambda t: (0, 0)),
        ],
        out_specs=pl.BlockSpec((TILE, 2), lambda t: (t, 0)),
        out_shape=jax.ShapeDtypeStruct((n, 2), jnp.float32),
    )
    out2 = jnp.stack([
        mlp_call(f[b], mxg[b], graph_w, graph_b.reshape(1, 2 * C), proj_w,
                 proj_b.reshape(1, C), ffn_w1, ffn_b1.reshape(1, 4 * C),
                 ffn_w2, ffn_b2.reshape(1, C), hw, hb)
        for b in range(B)
    ])

    return out2.transpose(0, 2, 1).reshape(B, 2, hp, wp)
